# baseline (speedup 1.0000x reference)
import jax
import jax.numpy as jnp
from jax.experimental import pallas as pl

_BM = 400
_BZ = 1000


def _z1_body(x_ref, w_ref, z_ref):
    z_ref[...] = jnp.dot(
        x_ref[...], w_ref[...], preferred_element_type=jnp.float32
    ).astype(jnp.bfloat16)


def _l1_body(adj_ref, z1_ref, b1_ref, w2_ref, h1_ref, z2_ref):
    ab = adj_ref[...].astype(jnp.bfloat16)
    h1 = jnp.dot(ab, z1_ref[...], preferred_element_type=jnp.float32) + b1_ref[...]
    h1_ref[...] = h1
    z2_ref[...] = jnp.dot(
        jnp.maximum(h1, 0.0).astype(jnp.bfloat16), w2_ref[...],
        preferred_element_type=jnp.float32,
    ).astype(jnp.bfloat16)


def _l3_body(adj_ref, z3_ref, b3_ref, h3_ref, out_ref):
    ab = adj_ref[...].astype(jnp.bfloat16)
    h3 = jnp.dot(ab, z3_ref[...], preferred_element_type=jnp.float32) + b3_ref[...]
    h3_ref[...] = h3
    m = jnp.max(h3, axis=1, keepdims=True)
    lse = jnp.log(jnp.sum(jnp.exp(h3 - m), axis=1, keepdims=True)) + m
    out_ref[...] = h3 - lse


def kernel(x, adj, W1, b1, W2, b2, W3, b3):
    n, nfeat = x.shape
    nhid = W1.shape[1]
    ncls = W3.shape[1]

    z1 = pl.pallas_call(
        _z1_body,
        grid=(n // _BZ,),
        in_specs=[
            pl.BlockSpec((_BZ, nfeat), lambda i: (i, 0)),
            pl.BlockSpec((nfeat, nhid), lambda i: (0, 0)),
        ],
        out_specs=pl.BlockSpec((_BZ, nhid), lambda i: (i, 0)),
        out_shape=jax.ShapeDtypeStruct((n, nhid), jnp.bfloat16),
    )(x, W1)

    h1, z2 = pl.pallas_call(
        _l1_body,
        grid=(n // _BM,),
        in_specs=[
            pl.BlockSpec((_BM, n), lambda i: (i, 0)),
            pl.BlockSpec((n, nhid), lambda i: (0, 0)),
            pl.BlockSpec((1, nhid), lambda i: (0, 0)),
            pl.BlockSpec((nhid, nhid), lambda i: (0, 0)),
        ],
        out_specs=[
            pl.BlockSpec((_BM, nhid), lambda i: (i, 0)),
            pl.BlockSpec((_BM, nhid), lambda i: (i, 0)),
        ],
        out_shape=[
            jax.ShapeDtypeStruct((n, nhid), jnp.float32),
            jax.ShapeDtypeStruct((n, nhid), jnp.bfloat16),
        ],
    )(adj, z1, b1.reshape(1, nhid), W2.astype(jnp.bfloat16))

    h2, z3 = pl.pallas_call(
        _l1_body,
        grid=(n // _BM,),
        in_specs=[
            pl.BlockSpec((_BM, n), lambda i: (i, 0)),
            pl.BlockSpec((n, nhid), lambda i: (0, 0)),
            pl.BlockSpec((1, nhid), lambda i: (0, 0)),
            pl.BlockSpec((nhid, ncls), lambda i: (0, 0)),
        ],
        out_specs=[
            pl.BlockSpec((_BM, nhid), lambda i: (i, 0)),
            pl.BlockSpec((_BM, ncls), lambda i: (i, 0)),
        ],
        out_shape=[
            jax.ShapeDtypeStruct((n, nhid), jnp.float32),
            jax.ShapeDtypeStruct((n, ncls), jnp.bfloat16),
        ],
    )(adj, z2, b2.reshape(1, nhid), W3.astype(jnp.bfloat16))

    h3, out = pl.pallas_call(
        _l3_body,
        grid=(n // _BM,),
        in_specs=[
            pl.BlockSpec((_BM, n), lambda i: (i, 0)),
            pl.BlockSpec((n, ncls), lambda i: (0, 0)),
            pl.BlockSpec((1, ncls), lambda i: (0, 0)),
        ],
        out_specs=[
            pl.BlockSpec((_BM, ncls), lambda i: (i, 0)),
            pl.BlockSpec((_BM, ncls), lambda i: (i, 0)),
        ],
        out_shape=[
            jax.ShapeDtypeStruct((n, ncls), jnp.float32),
            jax.ShapeDtypeStruct((n, ncls), jnp.float32),
        ],
    )(adj, z3, b3.reshape(1, ncls))

    return (out, h1, h2, h3)
